# baseline (speedup 1.0000x reference)
import functools

import jax
import jax.numpy as jnp
from jax import lax
from jax.experimental import pallas as pl
from jax.experimental.pallas import tpu as pltpu
from jax.experimental.pallas import tpu_sc as plsc

_B, _T, _C = 2, 2048, 768
_E = 64
_DFF = 768
_N = _B * _T
_K = 128

_L = 16
_NBKT = 2048
_CAP = 256


def _vsort_kv(k, v):
    return plsc.sort_key_val(k, v, descending=True)


def _minmax_kv(a, b):
    c = a[0] >= b[0]
    hi = (jnp.where(c, a[0], b[0]), jnp.where(c, a[1], b[1]))
    lo = (jnp.where(c, b[0], a[0]), jnp.where(c, b[1], a[1]))
    return hi, lo


def _rev_kv(a):
    return lax.rev(a[0], (0,)), lax.rev(a[1], (0,))


def _bitonic_clean(L):
    m = len(L)
    if m == 1:
        return [_vsort_kv(*L[0])]
    half = m // 2
    for i in range(half):
        hi, lo = _minmax_kv(L[i], L[i + half])
        L[i], L[i + half] = hi, lo
    return _bitonic_clean(L[:half]) + _bitonic_clean(L[half:])


def _bitonic_merge(A, B, top_only=False):
    w = len(A)
    H, Lo = [], []
    for i in range(w):
        hi, lo = _minmax_kv(A[i], _rev_kv(B[w - 1 - i]))
        H.append(hi)
        Lo.append(lo)
    return _bitonic_clean(H) if top_only else _bitonic_clean(H) + _bitonic_clean(Lo)


def _topk_row(e, pt_hbm, w_hbm, s_hbm, vals, hist, hist2, cand_v, cand_i, wstage, istage):
    lane = lax.iota(jnp.int32, _L)
    pltpu.sync_copy(pt_hbm.at[e], vals)

    def _zero(i, _):
        hist[pl.ds(i * _L, _L)] = jnp.zeros((_L,), jnp.int32)
        return 0

    lax.fori_loop(0, _NBKT * _L // _L, _zero, 0)

    def _hist(i, _):
        v = vals[pl.ds(i * _L, _L)]
        b = lax.shift_right_logical(plsc.bitcast(v, jnp.int32), 21)
        plsc.addupdate_scatter(hist, [b * _L + lane], jnp.ones((_L,), jnp.int32))
        return 0

    lax.fori_loop(0, _N // _L, _hist, 0)

    def _bsum(b):
        return jnp.sum(hist[pl.ds(b * _L, _L)])

    def _cond(c):
        b, acc = c
        return acc + _bsum(b) < _K

    def _body(c):
        b, acc = c
        return b - 1, acc + _bsum(b)

    bkt0, above = lax.while_loop(_cond, _body, (jnp.int32(_NBKT - 1), jnp.int32(0)))

    def _zero2(i, _):
        hist2[pl.ds(i * _L, _L)] = jnp.zeros((_L,), jnp.int32)
        return 0

    lax.fori_loop(0, 128, _zero2, 0)

    def _hist2(i, _):
        v = vals[pl.ds(i * _L, _L)]
        bits = plsc.bitcast(v, jnp.int32)
        m = lax.shift_right_logical(bits, 21) == bkt0
        sub = jnp.bitwise_and(lax.shift_right_logical(bits, 14), 127)
        plsc.addupdate_scatter(hist2, [sub * _L + lane],
                               jnp.ones((_L,), jnp.int32), mask=m)
        return 0

    lax.fori_loop(0, _N // _L, _hist2, 0)

    def _bsum2(s):
        return jnp.sum(hist2[pl.ds(s * _L, _L)])

    def _cond2(c):
        s, acc = c
        return acc + _bsum2(s) < _K

    def _body2(c):
        s, acc = c
        return s - 1, acc + _bsum2(s)

    sub1, _ = lax.while_loop(_cond2, _body2, (jnp.int32(127), above))
    t18 = bkt0 * 128 + sub1

    def _zcand(i, _):
        cand_v[pl.ds(i * _L, _L)] = jnp.zeros((_L,), jnp.float32)
        cand_i[pl.ds(i * _L, _L)] = jnp.zeros((_L,), jnp.int32)
        return 0

    lax.fori_loop(0, _CAP // _L, _zcand, 0)

    def _compact(i, off):
        v = vals[pl.ds(i * _L, _L)]
        b = lax.shift_right_logical(plsc.bitcast(v, jnp.int32), 14)
        m = b >= t18
        mi = m.astype(jnp.int32)
        pos = off + plsc.cumsum(mi) - 1
        plsc.store_scatter(cand_v, [pos], v, mask=m)
        plsc.store_scatter(cand_i, [pos], i * _L + lane, mask=m)
        return off + jnp.sum(mi)

    lax.fori_loop(0, _N // _L, _compact, jnp.int32(0))

    runs = [[_vsort_kv(cand_v[pl.ds(i * _L, _L)], cand_i[pl.ds(i * _L, _L)])]
            for i in range(_CAP // _L)]
    while len(runs) > 2:
        runs = [_bitonic_merge(runs[i], runs[i + 1]) for i in range(0, len(runs), 2)]
    top = _bitonic_merge(runs[0], runs[1], top_only=True)

    for i in range(_K // _L):
        wstage[pl.ds(i * _L, _L)] = top[i][0]
        istage[pl.ds(i * _L, _L)] = top[i][1]
    pltpu.sync_copy(wstage, w_hbm.at[e])
    pltpu.sync_copy(istage, s_hbm.at[e])


def _topk_sc_body(pt_hbm, w_hbm, s_hbm, vals, hist, hist2, cand_v, cand_i, wstage, istage):
    wid = lax.axis_index("s") * 2 + lax.axis_index("c")
    for j in range(2):
        _topk_row(wid * 2 + j, pt_hbm, w_hbm, s_hbm,
                  vals, hist, hist2, cand_v, cand_i, wstage, istage)


@functools.partial(jax.jit, static_argnames=("interpret",))
def _topk_sc(pt, interpret=False):
    f = pl.kernel(
        _topk_sc_body,
        interpret=interpret,
        out_type=(
            jax.ShapeDtypeStruct((_E, _K), jnp.float32),
            jax.ShapeDtypeStruct((_E, _K), jnp.int32),
        ),
        mesh=plsc.VectorSubcoreMesh(core_axis_name="c", subcore_axis_name="s"),
        compiler_params=pltpu.CompilerParams(needs_layout_passes=False),
        scratch_types=[
            pltpu.VMEM((_N,), jnp.float32),
            pltpu.VMEM((_NBKT * _L,), jnp.int32),
            pltpu.VMEM((128 * _L,), jnp.int32),
            pltpu.VMEM((_N,), jnp.float32),
            pltpu.VMEM((_N,), jnp.int32),
            pltpu.VMEM((_K,), jnp.float32),
            pltpu.VMEM((_K,), jnp.int32),
        ],
    )
    return f(pt)


def _moe_body(sel_smem, x_ref, w1_ref, w2_ref, wt_ref, out_ref, xs_ref, cs_ref):
    e = pl.program_id(0)

    @pl.when(e == 0)
    def _init():
        out_ref[...] = jnp.zeros_like(out_ref)

    def _gather(i, _):
        t = sel_smem[e * _K + i]
        xs_ref[pl.ds(i, 1), :] = x_ref[pl.ds(t, 1), :]
        return 0

    jax.lax.fori_loop(0, _K, _gather, 0, unroll=8)

    h = jnp.dot(xs_ref[...], w1_ref[0], preferred_element_type=jnp.float32)
    h = 0.5 * h * (1.0 + jax.lax.erf(h * 0.7071067811865476))
    out = jnp.dot(h, w2_ref[0], preferred_element_type=jnp.float32)
    lane = jax.lax.broadcasted_iota(jnp.int32, (_K, _E), 1)
    w_col = jnp.sum(jnp.where(lane == e, wt_ref[...], 0.0), axis=1, keepdims=True)
    cs_ref[...] = out * w_col

    def _scatter(i, _):
        t = sel_smem[e * _K + i]
        out_ref[pl.ds(t, 1), :] = out_ref[pl.ds(t, 1), :] + cs_ref[pl.ds(i, 1), :]
        return 0

    jax.lax.fori_loop(0, _K, _scatter, 0, unroll=8)


@functools.partial(jax.jit, static_argnames=("interpret",))
def _moe_pallas(x2d, w1, w2, weights_t, sel_flat, interpret=False):
    grid_spec = pltpu.PrefetchScalarGridSpec(
        num_scalar_prefetch=1,
        grid=(_E,),
        in_specs=[
            pl.BlockSpec((_N, _C), lambda e, sel: (0, 0)),
            pl.BlockSpec((1, _C, _DFF), lambda e, sel: (e, 0, 0)),
            pl.BlockSpec((1, _DFF, _C), lambda e, sel: (e, 0, 0)),
            pl.BlockSpec((_K, _E), lambda e, sel: (0, 0)),
        ],
        out_specs=pl.BlockSpec((_N, _C), lambda e, sel: (0, 0)),
        scratch_shapes=[
            pltpu.VMEM((_K, _C), jnp.float32),
            pltpu.VMEM((_K, _C), jnp.float32),
        ],
    )
    return pl.pallas_call(
        _moe_body,
        grid_spec=grid_spec,
        out_shape=jax.ShapeDtypeStruct((_N, _C), jnp.float32),
        compiler_params=pltpu.CompilerParams(
            dimension_semantics=("arbitrary",),
        ),
        interpret=interpret,
    )(sel_flat, x2d, w1, w2, weights_t)


def kernel(x, Wr, W1, W2):
    x2d = x.reshape(-1, _C)
    router_logits = x2d @ Wr.T
    probs = jax.nn.softmax(router_logits.astype(jnp.float32), axis=-1)
    weights, sel = _topk_sc(probs.T)
    results = _moe_pallas(
        x2d, W1, W2, weights.T, sel.reshape(-1).astype(jnp.int32)
    )
    return results.reshape(x.shape), router_logits, sel


# speedup vs baseline: 1.3789x; 1.3789x over previous
import functools

import jax
import jax.numpy as jnp
from jax import lax
from jax.experimental import pallas as pl
from jax.experimental.pallas import tpu as pltpu
from jax.experimental.pallas import tpu_sc as plsc

_B, _T, _C = 2, 2048, 768
_E = 64
_DFF = 768
_N = _B * _T
_K = 128

_L = 16
_NBKT = 2048
_CAP = 256


def _vsort_kv(k, v):
    lane = lax.iota(jnp.int32, _L)
    k1, v1 = plsc.sort_key_val(k, v, descending=True)
    sh = lax.gather(
        k1, jnp.maximum(lane - 1, 0)[:, None],
        lax.GatherDimensionNumbers(offset_dims=(), collapsed_slice_dims=(0,),
                                   start_index_map=(0,)),
        (1,), mode=lax.GatherScatterMode.PROMISE_IN_BOUNDS)
    eq = (k1 == sh) & (lane > 0)
    strict = plsc.cummax(jnp.where(eq, 0, lane))
    _, v2 = plsc.sort_key_val(strict * 4096 + v1, v1, descending=False)
    return k1, v2


def _minmax_kv(a, b):
    c = (a[0] > b[0]) | ((a[0] == b[0]) & (a[1] < b[1]))
    hi = (jnp.where(c, a[0], b[0]), jnp.where(c, a[1], b[1]))
    lo = (jnp.where(c, b[0], a[0]), jnp.where(c, b[1], a[1]))
    return hi, lo


def _rev_kv(a):
    return lax.rev(a[0], (0,)), lax.rev(a[1], (0,))


def _bitonic_clean(L):
    m = len(L)
    if m == 1:
        return [_vsort_kv(*L[0])]
    half = m // 2
    for i in range(half):
        hi, lo = _minmax_kv(L[i], L[i + half])
        L[i], L[i + half] = hi, lo
    return _bitonic_clean(L[:half]) + _bitonic_clean(L[half:])


def _bitonic_merge(A, B, top_only=False):
    w = len(A)
    H, Lo = [], []
    for i in range(w):
        hi, lo = _minmax_kv(A[i], _rev_kv(B[w - 1 - i]))
        H.append(hi)
        Lo.append(lo)
    return _bitonic_clean(H) if top_only else _bitonic_clean(H) + _bitonic_clean(Lo)


def _topk_row(e, pt_hbm, w_hbm, s_hbm, vals, hist, hist2, cand_v, cand_i, wstage, istage):
    lane = lax.iota(jnp.int32, _L)
    pltpu.sync_copy(pt_hbm.at[e], vals)

    def _zero(i, _):
        hist[pl.ds(i * _L, _L)] = jnp.zeros((_L,), jnp.int32)
        return 0

    lax.fori_loop(0, _NBKT * _L // _L, _zero, 0, unroll=8)

    def _hist(i, _):
        v = vals[pl.ds(i * _L, _L)]
        b = lax.shift_right_logical(plsc.bitcast(v, jnp.int32), 21)
        plsc.addupdate_scatter(hist, [b * _L + lane], jnp.ones((_L,), jnp.int32))
        return 0

    lax.fori_loop(0, _N // _L, _hist, 0, unroll=8)

    def _bsum(b):
        return jnp.sum(hist[pl.ds(b * _L, _L)])

    def _cond(c):
        b, acc = c
        return acc + _bsum(b) < _K

    def _body(c):
        b, acc = c
        return b - 1, acc + _bsum(b)

    bkt0, above = lax.while_loop(_cond, _body, (jnp.int32(507), jnp.int32(0)))

    def _zero2(i, _):
        hist2[pl.ds(i * _L, _L)] = jnp.zeros((_L,), jnp.int32)
        return 0

    lax.fori_loop(0, 128, _zero2, 0, unroll=8)

    def _hist2(i, _):
        v = vals[pl.ds(i * _L, _L)]
        bits = plsc.bitcast(v, jnp.int32)
        m = lax.shift_right_logical(bits, 21) == bkt0
        sub = jnp.bitwise_and(lax.shift_right_logical(bits, 14), 127)
        plsc.addupdate_scatter(hist2, [sub * _L + lane],
                               jnp.ones((_L,), jnp.int32), mask=m)
        return 0

    lax.fori_loop(0, _N // _L, _hist2, 0, unroll=8)

    def _bsum2(s):
        return jnp.sum(hist2[pl.ds(s * _L, _L)])

    def _cond2(c):
        s, acc = c
        return acc + _bsum2(s) < _K

    def _body2(c):
        s, acc = c
        return s - 1, acc + _bsum2(s)

    sub1, _ = lax.while_loop(_cond2, _body2, (jnp.int32(127), above))
    t18 = bkt0 * 128 + sub1

    def _zcand(i, _):
        cand_v[pl.ds(i * _L, _L)] = jnp.zeros((_L,), jnp.float32)
        cand_i[pl.ds(i * _L, _L)] = jnp.zeros((_L,), jnp.int32)
        return 0

    lax.fori_loop(0, _CAP // _L, _zcand, 0, unroll=8)

    def _compact(i, off):
        v = vals[pl.ds(i * _L, _L)]
        b = lax.shift_right_logical(plsc.bitcast(v, jnp.int32), 14)
        m = b >= t18
        mi = m.astype(jnp.int32)
        pos = off + plsc.cumsum(mi) - 1
        plsc.store_scatter(cand_v, [pos], v, mask=m)
        plsc.store_scatter(cand_i, [pos], i * _L + lane, mask=m)
        return off + jnp.sum(mi)

    lax.fori_loop(0, _N // _L, _compact, jnp.int32(0), unroll=4)

    runs = [[_vsort_kv(cand_v[pl.ds(i * _L, _L)],
                       jnp.int32(i * _L) + lane)]
            for i in range(_CAP // _L)]
    while len(runs) > 2:
        runs = [_bitonic_merge(runs[i], runs[i + 1]) for i in range(0, len(runs), 2)]
    top = _bitonic_merge(runs[0], runs[1], top_only=True)

    for i in range(_K // _L):
        wstage[pl.ds(i * _L, _L)] = top[i][0]
        istage[pl.ds(i * _L, _L)] = plsc.load_gather(cand_i, [top[i][1]])
    pltpu.sync_copy(wstage, w_hbm.at[e])
    pltpu.sync_copy(istage, s_hbm.at[e])


def _topk_sc_body(pt_hbm, w_hbm, s_hbm, vals, hist, hist2, cand_v, cand_i, wstage, istage):
    wid = lax.axis_index("s") * 2 + lax.axis_index("c")
    for j in range(2):
        _topk_row(wid * 2 + j, pt_hbm, w_hbm, s_hbm,
                  vals, hist, hist2, cand_v, cand_i, wstage, istage)


@functools.partial(jax.jit, static_argnames=("interpret",))
def _topk_sc(pt, interpret=False):
    f = pl.kernel(
        _topk_sc_body,
        interpret=interpret,
        out_type=(
            jax.ShapeDtypeStruct((_E, _K), jnp.float32),
            jax.ShapeDtypeStruct((_E, _K), jnp.int32),
        ),
        mesh=plsc.VectorSubcoreMesh(core_axis_name="c", subcore_axis_name="s"),
        compiler_params=pltpu.CompilerParams(needs_layout_passes=False),
        scratch_types=[
            pltpu.VMEM((_N,), jnp.float32),
            pltpu.VMEM((_NBKT * _L,), jnp.int32),
            pltpu.VMEM((128 * _L,), jnp.int32),
            pltpu.VMEM((_N,), jnp.float32),
            pltpu.VMEM((_N,), jnp.int32),
            pltpu.VMEM((_K,), jnp.float32),
            pltpu.VMEM((_K,), jnp.int32),
        ],
    )
    return f(pt)


def _moe_body(sel_smem, x_ref, w1_ref, w2_ref, wt_ref, out_ref, xs_ref, cs_ref):
    e = pl.program_id(0)

    @pl.when(e == 0)
    def _init():
        out_ref[...] = jnp.zeros_like(out_ref)

    def _gather(i, _):
        t = sel_smem[e * _K + i]
        xs_ref[pl.ds(i, 1), :] = x_ref[pl.ds(t, 1), :]
        return 0

    jax.lax.fori_loop(0, _K, _gather, 0, unroll=8)

    h = jnp.dot(xs_ref[...], w1_ref[0], preferred_element_type=jnp.float32)
    h = 0.5 * h * (1.0 + jax.lax.erf(h * 0.7071067811865476))
    out = jnp.dot(h, w2_ref[0], preferred_element_type=jnp.float32)
    lane = jax.lax.broadcasted_iota(jnp.int32, (_K, _E), 1)
    w_col = jnp.sum(jnp.where(lane == e, wt_ref[...], 0.0), axis=1, keepdims=True)
    cs_ref[...] = out * w_col

    def _scatter(i, _):
        t = sel_smem[e * _K + i]
        out_ref[pl.ds(t, 1), :] = out_ref[pl.ds(t, 1), :] + cs_ref[pl.ds(i, 1), :]
        return 0

    jax.lax.fori_loop(0, _K, _scatter, 0, unroll=8)


@functools.partial(jax.jit, static_argnames=("interpret",))
def _moe_pallas(x2d, w1, w2, weights_t, sel_flat, interpret=False):
    grid_spec = pltpu.PrefetchScalarGridSpec(
        num_scalar_prefetch=1,
        grid=(_E,),
        in_specs=[
            pl.BlockSpec((_N, _C), lambda e, sel: (0, 0)),
            pl.BlockSpec((1, _C, _DFF), lambda e, sel: (e, 0, 0)),
            pl.BlockSpec((1, _DFF, _C), lambda e, sel: (e, 0, 0)),
            pl.BlockSpec((_K, _E), lambda e, sel: (0, 0)),
        ],
        out_specs=pl.BlockSpec((_N, _C), lambda e, sel: (0, 0)),
        scratch_shapes=[
            pltpu.VMEM((_K, _C), jnp.float32),
            pltpu.VMEM((_K, _C), jnp.float32),
        ],
    )
    return pl.pallas_call(
        _moe_body,
        grid_spec=grid_spec,
        out_shape=jax.ShapeDtypeStruct((_N, _C), jnp.float32),
        compiler_params=pltpu.CompilerParams(
            dimension_semantics=("arbitrary",),
        ),
        interpret=interpret,
    )(sel_flat, x2d, w1, w2, weights_t)


def kernel(x, Wr, W1, W2):
    x2d = x.reshape(-1, _C)
    router_logits = x2d @ Wr.T
    probs = jax.nn.softmax(router_logits.astype(jnp.float32), axis=-1)
    weights, sel = _topk_sc(probs.T)
    results = _moe_pallas(
        x2d, W1, W2, weights.T, sel.reshape(-1).astype(jnp.int32)
    )
    return results.reshape(x.shape), router_logits, sel


# speedup vs baseline: 1.3954x; 1.0120x over previous
import functools

import jax
import jax.numpy as jnp
from jax import lax
from jax.experimental import pallas as pl
from jax.experimental.pallas import tpu as pltpu
from jax.experimental.pallas import tpu_sc as plsc

_B, _T, _C = 2, 2048, 768
_E = 64
_DFF = 768
_N = _B * _T
_K = 128

_L = 16
_NBKT = 512
_CAP = 256


def _vsort_kv(k, v):
    lane = lax.iota(jnp.int32, _L)
    k1, v1 = plsc.sort_key_val(k, v, descending=True)
    sh = lax.gather(
        k1, jnp.maximum(lane - 1, 0)[:, None],
        lax.GatherDimensionNumbers(offset_dims=(), collapsed_slice_dims=(0,),
                                   start_index_map=(0,)),
        (1,), mode=lax.GatherScatterMode.PROMISE_IN_BOUNDS)
    eq = (k1 == sh) & (lane > 0)
    strict = plsc.cummax(jnp.where(eq, 0, lane))
    _, v2 = plsc.sort_key_val(strict * 4096 + v1, v1, descending=False)
    return k1, v2


def _minmax_kv(a, b):
    c = (a[0] > b[0]) | ((a[0] == b[0]) & (a[1] < b[1]))
    hi = (jnp.where(c, a[0], b[0]), jnp.where(c, a[1], b[1]))
    lo = (jnp.where(c, b[0], a[0]), jnp.where(c, b[1], a[1]))
    return hi, lo


def _rev_kv(a):
    return lax.rev(a[0], (0,)), lax.rev(a[1], (0,))


def _bitonic_clean(L):
    m = len(L)
    if m == 1:
        return [_vsort_kv(*L[0])]
    half = m // 2
    for i in range(half):
        hi, lo = _minmax_kv(L[i], L[i + half])
        L[i], L[i + half] = hi, lo
    return _bitonic_clean(L[:half]) + _bitonic_clean(L[half:])


def _bitonic_merge(A, B, top_only=False):
    w = len(A)
    H, Lo = [], []
    for i in range(w):
        hi, lo = _minmax_kv(A[i], _rev_kv(B[w - 1 - i]))
        H.append(hi)
        Lo.append(lo)
    return _bitonic_clean(H) if top_only else _bitonic_clean(H) + _bitonic_clean(Lo)


def _topk_row(e, pt_hbm, w_hbm, s_hbm, vals, hist, hist2, cand_v, cand_i, wstage, istage):
    lane = lax.iota(jnp.int32, _L)

    def _zero(i, _):
        hist[pl.ds(i * _L, _L)] = jnp.zeros((_L,), jnp.int32)
        return 0

    lax.fori_loop(0, _NBKT * _L // _L, _zero, 0, unroll=8)

    def _hist(i, _):
        v = vals[pl.ds(i * _L, _L)]
        b = lax.shift_right_logical(plsc.bitcast(v, jnp.int32), 21)
        plsc.addupdate_scatter(hist, [b * _L + lane], jnp.ones((_L,), jnp.int32))
        return 0

    lax.fori_loop(0, _N // _L, _hist, 0, unroll=8)

    def _bsum(b):
        return jnp.sum(hist[pl.ds(b * _L, _L)])

    def _cond(c):
        b, acc = c
        return acc + _bsum(b) < _K

    def _body(c):
        b, acc = c
        return b - 1, acc + _bsum(b)

    bkt0, above = lax.while_loop(_cond, _body, (jnp.int32(507), jnp.int32(0)))

    def _zero2(i, _):
        hist2[pl.ds(i * _L, _L)] = jnp.zeros((_L,), jnp.int32)
        return 0

    lax.fori_loop(0, 128, _zero2, 0, unroll=8)

    def _hist2(i, _):
        v = vals[pl.ds(i * _L, _L)]
        bits = plsc.bitcast(v, jnp.int32)
        m = lax.shift_right_logical(bits, 21) == bkt0
        sub = jnp.bitwise_and(lax.shift_right_logical(bits, 14), 127)
        plsc.addupdate_scatter(hist2, [sub * _L + lane],
                               jnp.ones((_L,), jnp.int32), mask=m)
        return 0

    lax.fori_loop(0, _N // _L, _hist2, 0, unroll=8)

    def _bsum2(s):
        return jnp.sum(hist2[pl.ds(s * _L, _L)])

    def _cond2(c):
        s, acc = c
        return acc + _bsum2(s) < _K

    def _body2(c):
        s, acc = c
        return s - 1, acc + _bsum2(s)

    sub1, _ = lax.while_loop(_cond2, _body2, (jnp.int32(127), above))
    t18 = bkt0 * 128 + sub1

    def _zcand(i, _):
        cand_v[pl.ds(i * _L, _L)] = jnp.zeros((_L,), jnp.float32)
        cand_i[pl.ds(i * _L, _L)] = jnp.zeros((_L,), jnp.int32)
        return 0

    lax.fori_loop(0, _CAP // _L, _zcand, 0, unroll=8)

    def _compact(i, off):
        v = vals[pl.ds(i * _L, _L)]
        b = lax.shift_right_logical(plsc.bitcast(v, jnp.int32), 14)
        m = b >= t18
        mi = m.astype(jnp.int32)
        pos = off + plsc.cumsum(mi) - 1
        plsc.store_scatter(cand_v, [pos], v, mask=m)
        plsc.store_scatter(cand_i, [pos], i * _L + lane, mask=m)
        return off + jnp.sum(mi)

    lax.fori_loop(0, _N // _L, _compact, jnp.int32(0), unroll=4)

    runs = [[_vsort_kv(cand_v[pl.ds(i * _L, _L)],
                       jnp.int32(i * _L) + lane)]
            for i in range(_CAP // _L)]
    while len(runs) > 2:
        runs = [_bitonic_merge(runs[i], runs[i + 1]) for i in range(0, len(runs), 2)]
    top = _bitonic_merge(runs[0], runs[1], top_only=True)

    for i in range(_K // _L):
        wstage[pl.ds(i * _L, _L)] = top[i][0]
        istage[pl.ds(i * _L, _L)] = plsc.load_gather(cand_i, [top[i][1]])
    pltpu.sync_copy(wstage, w_hbm.at[e])
    pltpu.sync_copy(istage, s_hbm.at[e])


def _topk_sc_body(pt_hbm, w_hbm, s_hbm, vals, hist, hist2, cand_v, cand_i,
                  wstage, istage, sem0, sem1):
    wid = lax.axis_index("s") * 2 + lax.axis_index("c")
    cp0 = pltpu.async_copy(pt_hbm.at[2 * wid], vals.at[pl.ds(0, _N)], sem0)
    cp1 = pltpu.async_copy(pt_hbm.at[2 * wid + 1], vals.at[pl.ds(_N, _N)], sem1)
    cp0.wait()
    _topk_row(2 * wid, pt_hbm, w_hbm, s_hbm,
              vals.at[pl.ds(0, _N)], hist, hist2, cand_v, cand_i, wstage, istage)
    cp1.wait()
    _topk_row(2 * wid + 1, pt_hbm, w_hbm, s_hbm,
              vals.at[pl.ds(_N, _N)], hist, hist2, cand_v, cand_i, wstage, istage)


@functools.partial(jax.jit, static_argnames=("interpret",))
def _topk_sc(pt, interpret=False):
    f = pl.kernel(
        _topk_sc_body,
        interpret=interpret,
        out_type=(
            jax.ShapeDtypeStruct((_E, _K), jnp.float32),
            jax.ShapeDtypeStruct((_E, _K), jnp.int32),
        ),
        mesh=plsc.VectorSubcoreMesh(core_axis_name="c", subcore_axis_name="s"),
        compiler_params=pltpu.CompilerParams(needs_layout_passes=False),
        scratch_types=[
            pltpu.VMEM((2 * _N,), jnp.float32),
            pltpu.VMEM((_NBKT * _L,), jnp.int32),
            pltpu.VMEM((128 * _L,), jnp.int32),
            pltpu.VMEM((_N,), jnp.float32),
            pltpu.VMEM((_N,), jnp.int32),
            pltpu.VMEM((_K,), jnp.float32),
            pltpu.VMEM((_K,), jnp.int32),
            pltpu.SemaphoreType.DMA,
            pltpu.SemaphoreType.DMA,
        ],
    )
    return f(pt)


def _moe_body(sel_smem, x_ref, w1_ref, w2_ref, wt_ref, out_ref, xs_ref, cs_ref):
    e = pl.program_id(0)

    @pl.when(e == 0)
    def _init():
        out_ref[...] = jnp.zeros_like(out_ref)

    def _gather(i, _):
        t = sel_smem[e * _K + i]
        xs_ref[pl.ds(i, 1), :] = x_ref[pl.ds(t, 1), :]
        return 0

    jax.lax.fori_loop(0, _K, _gather, 0, unroll=8)

    h = jnp.dot(xs_ref[...], w1_ref[0], preferred_element_type=jnp.float32)
    h = 0.5 * h * (1.0 + jax.lax.erf(h * 0.7071067811865476))
    out = jnp.dot(h, w2_ref[0], preferred_element_type=jnp.float32)
    lane = jax.lax.broadcasted_iota(jnp.int32, (_K, _E), 1)
    w_col = jnp.sum(jnp.where(lane == e, wt_ref[...], 0.0), axis=1, keepdims=True)
    cs_ref[...] = out * w_col

    def _scatter(i, _):
        t = sel_smem[e * _K + i]
        out_ref[pl.ds(t, 1), :] = out_ref[pl.ds(t, 1), :] + cs_ref[pl.ds(i, 1), :]
        return 0

    jax.lax.fori_loop(0, _K, _scatter, 0, unroll=8)


@functools.partial(jax.jit, static_argnames=("interpret",))
def _moe_pallas(x2d, w1, w2, weights_t, sel_flat, interpret=False):
    grid_spec = pltpu.PrefetchScalarGridSpec(
        num_scalar_prefetch=1,
        grid=(_E,),
        in_specs=[
            pl.BlockSpec((_N, _C), lambda e, sel: (0, 0)),
            pl.BlockSpec((1, _C, _DFF), lambda e, sel: (e, 0, 0)),
            pl.BlockSpec((1, _DFF, _C), lambda e, sel: (e, 0, 0)),
            pl.BlockSpec((_K, _E), lambda e, sel: (0, 0)),
        ],
        out_specs=pl.BlockSpec((_N, _C), lambda e, sel: (0, 0)),
        scratch_shapes=[
            pltpu.VMEM((_K, _C), jnp.float32),
            pltpu.VMEM((_K, _C), jnp.float32),
        ],
    )
    return pl.pallas_call(
        _moe_body,
        grid_spec=grid_spec,
        out_shape=jax.ShapeDtypeStruct((_N, _C), jnp.float32),
        compiler_params=pltpu.CompilerParams(
            dimension_semantics=("arbitrary",),
        ),
        interpret=interpret,
    )(sel_flat, x2d, w1, w2, weights_t)


def kernel(x, Wr, W1, W2):
    x2d = x.reshape(-1, _C)
    router_logits = x2d @ Wr.T
    probs = jax.nn.softmax(router_logits.astype(jnp.float32), axis=-1)
    weights, sel = _topk_sc(probs.T)
    results = _moe_pallas(
        x2d, W1, W2, weights.T, sel.reshape(-1).astype(jnp.int32)
    )
    return results.reshape(x.shape), router_logits, sel


# speedup vs baseline: 1.3966x; 1.0008x over previous
import functools

import jax
import jax.numpy as jnp
from jax import lax
from jax.experimental import pallas as pl
from jax.experimental.pallas import tpu as pltpu
from jax.experimental.pallas import tpu_sc as plsc

_B, _T, _C = 2, 2048, 768
_E = 64
_DFF = 768
_N = _B * _T
_K = 128

_L = 16
_NBKT = 512
_CAP = 256


def _vsort_kv(k, v):
    lane = lax.iota(jnp.int32, _L)
    k1, v1 = plsc.sort_key_val(k, v, descending=True)
    sh = lax.gather(
        k1, jnp.maximum(lane - 1, 0)[:, None],
        lax.GatherDimensionNumbers(offset_dims=(), collapsed_slice_dims=(0,),
                                   start_index_map=(0,)),
        (1,), mode=lax.GatherScatterMode.PROMISE_IN_BOUNDS)
    eq = (k1 == sh) & (lane > 0)
    strict = plsc.cummax(jnp.where(eq, 0, lane))
    _, v2 = plsc.sort_key_val(strict * 4096 + v1, v1, descending=False)
    return k1, v2


def _minmax_kv(a, b):
    c = (a[0] > b[0]) | ((a[0] == b[0]) & (a[1] < b[1]))
    hi = (jnp.where(c, a[0], b[0]), jnp.where(c, a[1], b[1]))
    lo = (jnp.where(c, b[0], a[0]), jnp.where(c, b[1], a[1]))
    return hi, lo


def _rev_kv(a):
    return lax.rev(a[0], (0,)), lax.rev(a[1], (0,))


def _bitonic_clean(L):
    m = len(L)
    if m == 1:
        return [_vsort_kv(*L[0])]
    half = m // 2
    for i in range(half):
        hi, lo = _minmax_kv(L[i], L[i + half])
        L[i], L[i + half] = hi, lo
    return _bitonic_clean(L[:half]) + _bitonic_clean(L[half:])


def _bitonic_merge(A, B, top_only=False):
    w = len(A)
    H, Lo = [], []
    for i in range(w):
        hi, lo = _minmax_kv(A[i], _rev_kv(B[w - 1 - i]))
        H.append(hi)
        Lo.append(lo)
    return _bitonic_clean(H) if top_only else _bitonic_clean(H) + _bitonic_clean(Lo)


def _topk_row(e, pt_hbm, w_hbm, s_hbm, vals, hist, hist2, cand_v, cand_i, wstage, istage):
    lane = lax.iota(jnp.int32, _L)

    def _zero(i, _):
        hist[pl.ds(i * _L, _L)] = jnp.zeros((_L,), jnp.int32)
        return 0

    lax.fori_loop(0, _NBKT * _L // _L, _zero, 0, unroll=8)

    def _hist(i, _):
        v = vals[pl.ds(i * _L, _L)]
        b = lax.shift_right_logical(plsc.bitcast(v, jnp.int32), 21)
        plsc.addupdate_scatter(hist, [b * _L + lane], jnp.ones((_L,), jnp.int32))
        return 0

    lax.fori_loop(0, _N // _L, _hist, 0, unroll=8)

    def _bsum(b):
        return jnp.sum(hist[pl.ds(b * _L, _L)])

    def _cond(c):
        b, acc = c
        return acc + _bsum(b) < _K

    def _body(c):
        b, acc = c
        return b - 1, acc + _bsum(b)

    bkt0, above = lax.while_loop(_cond, _body, (jnp.int32(507), jnp.int32(0)))

    def _zero2(i, _):
        hist2[pl.ds(i * _L, _L)] = jnp.zeros((_L,), jnp.int32)
        return 0

    lax.fori_loop(0, 128, _zero2, 0, unroll=8)

    def _hist2(i, _):
        v = vals[pl.ds(i * _L, _L)]
        bits = plsc.bitcast(v, jnp.int32)
        m = lax.shift_right_logical(bits, 21) == bkt0
        sub = jnp.bitwise_and(lax.shift_right_logical(bits, 14), 127)
        plsc.addupdate_scatter(hist2, [sub * _L + lane],
                               jnp.ones((_L,), jnp.int32), mask=m)
        return 0

    lax.fori_loop(0, _N // _L, _hist2, 0, unroll=8)

    def _bsum2(s):
        return jnp.sum(hist2[pl.ds(s * _L, _L)])

    def _cond2(c):
        s, acc = c
        return acc + _bsum2(s) < _K

    def _body2(c):
        s, acc = c
        return s - 1, acc + _bsum2(s)

    sub1, _ = lax.while_loop(_cond2, _body2, (jnp.int32(127), above))
    t18 = bkt0 * 128 + sub1

    def _zcand(i, _):
        cand_v[pl.ds(i * _L, _L)] = jnp.zeros((_L,), jnp.float32)
        cand_i[pl.ds(i * _L, _L)] = jnp.zeros((_L,), jnp.int32)
        return 0

    lax.fori_loop(0, _CAP // _L, _zcand, 0, unroll=8)

    def _compact(i, off):
        v = vals[pl.ds(i * _L, _L)]
        b = lax.shift_right_logical(plsc.bitcast(v, jnp.int32), 14)
        m = b >= t18
        mi = m.astype(jnp.int32)
        pos = off + plsc.cumsum(mi) - 1
        plsc.store_scatter(cand_v, [pos], v, mask=m)
        plsc.store_scatter(cand_i, [pos], i * _L + lane, mask=m)
        return off + plsc.all_reduce_population_count(m)

    lax.fori_loop(0, _N // _L, _compact, jnp.zeros((_L,), jnp.int32), unroll=4)

    runs = [[_vsort_kv(cand_v[pl.ds(i * _L, _L)],
                       jnp.int32(i * _L) + lane)]
            for i in range(_CAP // _L)]
    while len(runs) > 2:
        runs = [_bitonic_merge(runs[i], runs[i + 1]) for i in range(0, len(runs), 2)]
    top = _bitonic_merge(runs[0], runs[1], top_only=True)

    for i in range(_K // _L):
        wstage[pl.ds(i * _L, _L)] = top[i][0]
        istage[pl.ds(i * _L, _L)] = plsc.load_gather(cand_i, [top[i][1]])
    pltpu.sync_copy(wstage, w_hbm.at[e])
    pltpu.sync_copy(istage, s_hbm.at[e])


def _topk_sc_body(pt_hbm, w_hbm, s_hbm, vals, hist, hist2, cand_v, cand_i,
                  wstage, istage, sem0, sem1):
    wid = lax.axis_index("s") * 2 + lax.axis_index("c")
    cp0 = pltpu.async_copy(pt_hbm.at[2 * wid], vals.at[pl.ds(0, _N)], sem0)
    cp1 = pltpu.async_copy(pt_hbm.at[2 * wid + 1], vals.at[pl.ds(_N, _N)], sem1)
    cp0.wait()
    _topk_row(2 * wid, pt_hbm, w_hbm, s_hbm,
              vals.at[pl.ds(0, _N)], hist, hist2, cand_v, cand_i, wstage, istage)
    cp1.wait()
    _topk_row(2 * wid + 1, pt_hbm, w_hbm, s_hbm,
              vals.at[pl.ds(_N, _N)], hist, hist2, cand_v, cand_i, wstage, istage)


@functools.partial(jax.jit, static_argnames=("interpret",))
def _topk_sc(pt, interpret=False):
    f = pl.kernel(
        _topk_sc_body,
        interpret=interpret,
        out_type=(
            jax.ShapeDtypeStruct((_E, _K), jnp.float32),
            jax.ShapeDtypeStruct((_E, _K), jnp.int32),
        ),
        mesh=plsc.VectorSubcoreMesh(core_axis_name="c", subcore_axis_name="s"),
        compiler_params=pltpu.CompilerParams(needs_layout_passes=False),
        scratch_types=[
            pltpu.VMEM((2 * _N,), jnp.float32),
            pltpu.VMEM((_NBKT * _L,), jnp.int32),
            pltpu.VMEM((128 * _L,), jnp.int32),
            pltpu.VMEM((_N,), jnp.float32),
            pltpu.VMEM((_N,), jnp.int32),
            pltpu.VMEM((_K,), jnp.float32),
            pltpu.VMEM((_K,), jnp.int32),
            pltpu.SemaphoreType.DMA,
            pltpu.SemaphoreType.DMA,
        ],
    )
    return f(pt)


def _moe_body(sel_smem, x_ref, w1_ref, w2_ref, wt_ref, out_ref, xs_ref, cs_ref):
    e = pl.program_id(0)

    @pl.when(e == 0)
    def _init():
        out_ref[...] = jnp.zeros_like(out_ref)

    def _gather(i, _):
        t = sel_smem[e * _K + i]
        xs_ref[pl.ds(i, 1), :] = x_ref[pl.ds(t, 1), :]
        return 0

    jax.lax.fori_loop(0, _K, _gather, 0, unroll=8)

    h = jnp.dot(xs_ref[...], w1_ref[0], preferred_element_type=jnp.float32)
    h = 0.5 * h * (1.0 + jax.lax.erf(h * 0.7071067811865476))
    out = jnp.dot(h, w2_ref[0], preferred_element_type=jnp.float32)
    lane = jax.lax.broadcasted_iota(jnp.int32, (_K, _E), 1)
    w_col = jnp.sum(jnp.where(lane == e, wt_ref[...], 0.0), axis=1, keepdims=True)
    cs_ref[...] = out * w_col

    def _scatter(i, _):
        t = sel_smem[e * _K + i]
        out_ref[pl.ds(t, 1), :] = out_ref[pl.ds(t, 1), :] + cs_ref[pl.ds(i, 1), :]
        return 0

    jax.lax.fori_loop(0, _K, _scatter, 0, unroll=8)


@functools.partial(jax.jit, static_argnames=("interpret",))
def _moe_pallas(x2d, w1, w2, weights_t, sel_flat, interpret=False):
    grid_spec = pltpu.PrefetchScalarGridSpec(
        num_scalar_prefetch=1,
        grid=(_E,),
        in_specs=[
            pl.BlockSpec((_N, _C), lambda e, sel: (0, 0)),
            pl.BlockSpec((1, _C, _DFF), lambda e, sel: (e, 0, 0)),
            pl.BlockSpec((1, _DFF, _C), lambda e, sel: (e, 0, 0)),
            pl.BlockSpec((_K, _E), lambda e, sel: (0, 0)),
        ],
        out_specs=pl.BlockSpec((_N, _C), lambda e, sel: (0, 0)),
        scratch_shapes=[
            pltpu.VMEM((_K, _C), jnp.float32),
            pltpu.VMEM((_K, _C), jnp.float32),
        ],
    )
    return pl.pallas_call(
        _moe_body,
        grid_spec=grid_spec,
        out_shape=jax.ShapeDtypeStruct((_N, _C), jnp.float32),
        compiler_params=pltpu.CompilerParams(
            dimension_semantics=("arbitrary",),
        ),
        interpret=interpret,
    )(sel_flat, x2d, w1, w2, weights_t)


def kernel(x, Wr, W1, W2):
    x2d = x.reshape(-1, _C)
    router_logits = x2d @ Wr.T
    probs = jax.nn.softmax(router_logits.astype(jnp.float32), axis=-1)
    weights, sel = _topk_sc(probs.T)
    results = _moe_pallas(
        x2d, W1, W2, weights.T, sel.reshape(-1).astype(jnp.int32)
    )
    return results.reshape(x.shape), router_logits, sel


# speedup vs baseline: 1.3975x; 1.0007x over previous
import functools

import jax
import jax.numpy as jnp
from jax import lax
from jax.experimental import pallas as pl
from jax.experimental.pallas import tpu as pltpu
from jax.experimental.pallas import tpu_sc as plsc

_B, _T, _C = 2, 2048, 768
_E = 64
_DFF = 768
_N = _B * _T
_K = 128

_L = 16
_NBKT = 512
_CAP = 256


def _vsort_kv(k, v):
    lane = lax.iota(jnp.int32, _L)
    k1, v1 = plsc.sort_key_val(k, v, descending=True)
    sh = lax.gather(
        k1, jnp.maximum(lane - 1, 0)[:, None],
        lax.GatherDimensionNumbers(offset_dims=(), collapsed_slice_dims=(0,),
                                   start_index_map=(0,)),
        (1,), mode=lax.GatherScatterMode.PROMISE_IN_BOUNDS)
    eq = (k1 == sh) & (lane > 0)
    strict = plsc.cummax(jnp.where(eq, 0, lane))
    _, v2 = plsc.sort_key_val(strict * 4096 + v1, v1, descending=False)
    return k1, v2


def _minmax_kv(a, b):
    c = (a[0] > b[0]) | ((a[0] == b[0]) & (a[1] < b[1]))
    hi = (jnp.where(c, a[0], b[0]), jnp.where(c, a[1], b[1]))
    lo = (jnp.where(c, b[0], a[0]), jnp.where(c, b[1], a[1]))
    return hi, lo


def _rev_kv(a):
    return lax.rev(a[0], (0,)), lax.rev(a[1], (0,))


def _bitonic_clean(L):
    m = len(L)
    if m == 1:
        return [_vsort_kv(*L[0])]
    half = m // 2
    for i in range(half):
        hi, lo = _minmax_kv(L[i], L[i + half])
        L[i], L[i + half] = hi, lo
    return _bitonic_clean(L[:half]) + _bitonic_clean(L[half:])


def _bitonic_merge(A, B, top_only=False):
    w = len(A)
    H, Lo = [], []
    for i in range(w):
        hi, lo = _minmax_kv(A[i], _rev_kv(B[w - 1 - i]))
        H.append(hi)
        Lo.append(lo)
    return _bitonic_clean(H) if top_only else _bitonic_clean(H) + _bitonic_clean(Lo)


def _topk_row(e, pt_hbm, w_hbm, s_hbm, vals, hist, hist2, cand_v, cand_i, wstage, istage):
    lane = lax.iota(jnp.int32, _L)

    def _zero(i, _):
        hist[pl.ds(i * _L, _L)] = jnp.zeros((_L,), jnp.int32)
        return 0

    lax.fori_loop(0, _NBKT * _L // _L, _zero, 0, unroll=8)

    def _hist(i, _):
        v = vals[pl.ds(i * _L, _L)]
        b = lax.shift_right_logical(plsc.bitcast(v, jnp.int32), 21)
        plsc.addupdate_scatter(hist, [b * _L + lane], jnp.ones((_L,), jnp.int32))
        return 0

    lax.fori_loop(0, _N // _L, _hist, 0, unroll=8)

    def _bsum(b):
        return jnp.sum(hist[pl.ds(b * _L, _L)])

    def _cond(c):
        b, acc = c
        return acc + _bsum(b) < _K

    def _body(c):
        b, acc = c
        return b - 1, acc + _bsum(b)

    bkt0, above = lax.while_loop(_cond, _body, (jnp.int32(507), jnp.int32(0)))

    def _zero2(i, _):
        hist2[pl.ds(i * _L, _L)] = jnp.zeros((_L,), jnp.int32)
        return 0

    lax.fori_loop(0, 128, _zero2, 0, unroll=8)

    def _hist2(i, _):
        v = vals[pl.ds(i * _L, _L)]
        bits = plsc.bitcast(v, jnp.int32)
        m = lax.shift_right_logical(bits, 21) == bkt0
        sub = jnp.bitwise_and(lax.shift_right_logical(bits, 14), 127)
        plsc.addupdate_scatter(hist2, [sub * _L + lane],
                               jnp.ones((_L,), jnp.int32), mask=m)
        return 0

    lax.fori_loop(0, _N // _L, _hist2, 0, unroll=8)

    def _bsum2(s):
        return jnp.sum(hist2[pl.ds(s * _L, _L)])

    def _cond2(c):
        s, acc = c
        return acc + _bsum2(s) < _K

    def _body2(c):
        s, acc = c
        return s - 1, acc + _bsum2(s)

    sub1, _ = lax.while_loop(_cond2, _body2, (jnp.int32(127), above))
    t18 = bkt0 * 128 + sub1

    def _zcand(i, _):
        cand_v[pl.ds(i * _L, _L)] = jnp.zeros((_L,), jnp.float32)
        cand_i[pl.ds(i * _L, _L)] = jnp.zeros((_L,), jnp.int32)
        return 0

    lax.fori_loop(0, _CAP // _L, _zcand, 0, unroll=8)

    def _compact(i, off):
        v = vals[pl.ds(i * _L, _L)]
        b = lax.shift_right_logical(plsc.bitcast(v, jnp.int32), 14)
        m = b >= t18
        mi = m.astype(jnp.int32)
        pos = off + plsc.cumsum(mi) - 1
        plsc.store_scatter(cand_v, [pos], v, mask=m)
        plsc.store_scatter(cand_i, [pos], i * _L + lane, mask=m)
        return off + plsc.all_reduce_population_count(m)

    lax.fori_loop(0, _N // _L, _compact, jnp.zeros((_L,), jnp.int32), unroll=4)

    runs = [[_vsort_kv(cand_v[pl.ds(i * _L, _L)],
                       jnp.int32(i * _L) + lane)]
            for i in range(_CAP // _L)]
    while len(runs) > 2:
        runs = [_bitonic_merge(runs[i], runs[i + 1]) for i in range(0, len(runs), 2)]
    top = _bitonic_merge(runs[0], runs[1], top_only=True)

    for i in range(_K // _L):
        wstage[pl.ds(i * _L, _L)] = top[i][0]
        istage[pl.ds(i * _L, _L)] = plsc.load_gather(cand_i, [top[i][1]])
    pltpu.sync_copy(wstage, w_hbm.at[e])
    pltpu.sync_copy(istage, s_hbm.at[e])


def _topk_sc_body(pt_hbm, w_hbm, s_hbm, vals, hist, hist2, cand_v, cand_i,
                  wstage, istage, sem0, sem1):
    wid = lax.axis_index("s") * 2 + lax.axis_index("c")
    cp0 = pltpu.async_copy(pt_hbm.at[2 * wid], vals.at[pl.ds(0, _N)], sem0)
    cp1 = pltpu.async_copy(pt_hbm.at[2 * wid + 1], vals.at[pl.ds(_N, _N)], sem1)
    cp0.wait()
    _topk_row(2 * wid, pt_hbm, w_hbm, s_hbm,
              vals.at[pl.ds(0, _N)], hist, hist2, cand_v, cand_i, wstage, istage)
    cp1.wait()
    _topk_row(2 * wid + 1, pt_hbm, w_hbm, s_hbm,
              vals.at[pl.ds(_N, _N)], hist, hist2, cand_v, cand_i, wstage, istage)


@functools.partial(jax.jit, static_argnames=("interpret",))
def _topk_sc(pt, interpret=False):
    f = pl.kernel(
        _topk_sc_body,
        interpret=interpret,
        out_type=(
            jax.ShapeDtypeStruct((_E, _K), jnp.float32),
            jax.ShapeDtypeStruct((_E, _K), jnp.int32),
        ),
        mesh=plsc.VectorSubcoreMesh(core_axis_name="c", subcore_axis_name="s"),
        compiler_params=pltpu.CompilerParams(needs_layout_passes=False),
        scratch_types=[
            pltpu.VMEM((2 * _N,), jnp.float32),
            pltpu.VMEM((_NBKT * _L,), jnp.int32),
            pltpu.VMEM((128 * _L,), jnp.int32),
            pltpu.VMEM((_N,), jnp.float32),
            pltpu.VMEM((_N,), jnp.int32),
            pltpu.VMEM((_K,), jnp.float32),
            pltpu.VMEM((_K,), jnp.int32),
            pltpu.SemaphoreType.DMA,
            pltpu.SemaphoreType.DMA,
        ],
    )
    return f(pt)


def _moe_body(sel_smem, x_ref, w1_ref, w2_ref, wt_ref, out_ref, xs_ref, cs_ref):
    e = pl.program_id(0)

    @pl.when(e == 0)
    def _init():
        out_ref[...] = jnp.zeros_like(out_ref)

    def _gather(i, _):
        t = sel_smem[e * _K + i]
        xs_ref[pl.ds(i, 1), :] = x_ref[pl.ds(t, 1), :]
        return 0

    jax.lax.fori_loop(0, _K, _gather, 0, unroll=16)

    h = jnp.dot(xs_ref[...], w1_ref[0], preferred_element_type=jnp.float32)
    h = 0.5 * h * (1.0 + jax.lax.erf(h * 0.7071067811865476))
    out = jnp.dot(h, w2_ref[0], preferred_element_type=jnp.float32)
    lane = jax.lax.broadcasted_iota(jnp.int32, (_K, _E), 1)
    w_col = jnp.sum(jnp.where(lane == e, wt_ref[...], 0.0), axis=1, keepdims=True)
    cs_ref[...] = out * w_col

    def _scatter(i, _):
        t = sel_smem[e * _K + i]
        out_ref[pl.ds(t, 1), :] = out_ref[pl.ds(t, 1), :] + cs_ref[pl.ds(i, 1), :]
        return 0

    jax.lax.fori_loop(0, _K, _scatter, 0, unroll=16)


@functools.partial(jax.jit, static_argnames=("interpret",))
def _moe_pallas(x2d, w1, w2, weights_t, sel_flat, interpret=False):
    grid_spec = pltpu.PrefetchScalarGridSpec(
        num_scalar_prefetch=1,
        grid=(_E,),
        in_specs=[
            pl.BlockSpec((_N, _C), lambda e, sel: (0, 0)),
            pl.BlockSpec((1, _C, _DFF), lambda e, sel: (e, 0, 0)),
            pl.BlockSpec((1, _DFF, _C), lambda e, sel: (e, 0, 0)),
            pl.BlockSpec((_K, _E), lambda e, sel: (0, 0)),
        ],
        out_specs=pl.BlockSpec((_N, _C), lambda e, sel: (0, 0)),
        scratch_shapes=[
            pltpu.VMEM((_K, _C), jnp.float32),
            pltpu.VMEM((_K, _C), jnp.float32),
        ],
    )
    return pl.pallas_call(
        _moe_body,
        grid_spec=grid_spec,
        out_shape=jax.ShapeDtypeStruct((_N, _C), jnp.float32),
        compiler_params=pltpu.CompilerParams(
            dimension_semantics=("arbitrary",),
        ),
        interpret=interpret,
    )(sel_flat, x2d, w1, w2, weights_t)


def kernel(x, Wr, W1, W2):
    x2d = x.reshape(-1, _C)
    router_logits = x2d @ Wr.T
    probs = jax.nn.softmax(router_logits.astype(jnp.float32), axis=-1)
    weights, sel = _topk_sc(probs.T)
    results = _moe_pallas(
        x2d, W1, W2, weights.T, sel.reshape(-1).astype(jnp.int32)
    )
    return results.reshape(x.shape), router_logits, sel
